# Initial kernel scaffold; baseline (speedup 1.0000x reference)
import jax
import jax.numpy as jnp
from jax.experimental import pallas as pl


def kernel(x, edge_index, W1, b1, W2, b2, W3, b3, W4, b4, W5, b5, fc1_W, fc1_b, fc2_W, fc2_b, fc3_W, fc3_b, fc4_W, fc4_b, fc5_W, fc5_b):
    raise NotImplementedError("write your pallas kernel here")


# speedup vs baseline: 12.0100x; 12.0100x over previous
import functools

import jax
import jax.numpy as jnp
from jax import lax
from jax.experimental import pallas as pl
from jax.experimental.pallas import tpu as pltpu
from jax.experimental.pallas import tpu_sc as plsc

N = 10000
E = 320000
F = 128
NP = 10240
CH = 128
NCH = 79
W = 32
EP = W * NCH * CH
SPAN = NP // 16
ZR = SPAN // CH

def _degree_body(dst_hbm, zeros1_hbm, ones1_hbm, out_hbm, idx_d, zb, ob, deg_sh):
    c = lax.axis_index("c")
    s = lax.axis_index("s")
    wid = c * 16 + s
    pltpu.sync_copy(dst_hbm.at[wid], idx_d)
    pltpu.sync_copy(zeros1_hbm, zb)
    pltpu.sync_copy(ones1_hbm, ob)
    for z in range(ZR):
        pltpu.sync_copy(zb, deg_sh.at[pl.ds(s * SPAN + z * CH, CH)])
    plsc.subcore_barrier()

    def chunk(j, carry):
        pltpu.sync_copy(ob, deg_sh.at[idx_d.at[j]], add=True)
        return carry

    lax.fori_loop(0, NCH, chunk, 0)
    plsc.subcore_barrier()
    for z in range(ZR):
        pltpu.sync_copy(deg_sh.at[pl.ds(s * SPAN + z * CH, CH)], zb)
        pltpu.sync_copy(zb, out_hbm.at[c, pl.ds(s * SPAN + z * CH, CH)])


def _scatter_body(hs_hbm, src_hbm, dst_hbm, zeros2_hbm, out_hbm,
                  idx_s, idx_d, rows, acc, sem):
    c = lax.axis_index("c")
    s = lax.axis_index("s")
    wid = c * 16 + s
    pltpu.sync_copy(src_hbm.at[wid], idx_s)
    pltpu.sync_copy(dst_hbm.at[wid], idx_d)
    pltpu.sync_copy(zeros2_hbm, rows)
    for z in range(ZR):
        pltpu.sync_copy(rows, acc.at[pl.ds(s * SPAN + z * CH, CH)])
    plsc.subcore_barrier()

    def chunk(j, carry):
        pltpu.async_copy(hs_hbm.at[idx_s.at[j]], rows, sem).wait()
        pltpu.sync_copy(rows, acc.at[idx_d.at[j]], add=True)
        return carry

    lax.fori_loop(0, NCH, chunk, 0)
    plsc.subcore_barrier()
    for z in range(ZR):
        pltpu.sync_copy(acc.at[pl.ds(s * SPAN + z * CH, CH)], rows)
        pltpu.sync_copy(rows, out_hbm.at[c, pl.ds(s * SPAN + z * CH, CH)])


@functools.cache
def _sc_kernels():
    mesh = plsc.VectorSubcoreMesh(core_axis_name="c", subcore_axis_name="s")
    deg = pl.kernel(
        _degree_body,
        mesh=mesh,
        out_type=jax.ShapeDtypeStruct((2, NP), jnp.float32),
        scratch_types=[
            pltpu.VMEM((NCH, CH), jnp.int32),
            pltpu.VMEM((CH,), jnp.float32),
            pltpu.VMEM((CH,), jnp.float32),
            pltpu.VMEM_SHARED((NP,), jnp.float32),
        ],
    )
    scat = pl.kernel(
        _scatter_body,
        mesh=mesh,
        out_type=jax.ShapeDtypeStruct((2, NP, F), jnp.float32),
        scratch_types=[
            pltpu.VMEM((NCH, CH), jnp.int32),
            pltpu.VMEM((NCH, CH), jnp.int32),
            pltpu.VMEM((CH, F), jnp.float32),
            pltpu.VMEM_SHARED((NP, F), jnp.float32),
            pltpu.SemaphoreType.DMA,
        ],
    )
    return deg, scat


_BLK = 2048
_GRID = NP // _BLK


def _row_spec(w=F):
    return pl.BlockSpec((_BLK, w), lambda i: (i, 0))


def _full_spec(*shape):
    nd = len(shape)
    return pl.BlockSpec(shape, lambda i, _n=nd: (0,) * nd)


def _vec_spec():
    return pl.BlockSpec((_BLK,), lambda i: (i,))


def _tc_mm1_body(x_ref, w1_ref, dp_ref, hs1_ref, dinv_ref):
    deg = dp_ref[0, :] + dp_ref[1, :] + 1.0
    dinv = lax.rsqrt(deg)
    h = jnp.dot(x_ref[...], w1_ref[...], preferred_element_type=jnp.float32)
    hs1_ref[...] = h * dinv[:, None]
    dinv_ref[...] = dinv


def _tc_mid_body(p0_ref, p1_ref, hs_ref, dinv_ref, b_ref, x_ref,
                 wa_ref, wb_ref, out_ref):
    dinv = dinv_ref[...]
    h1r = jnp.maximum(
        dinv[:, None] * (p0_ref[...] + p1_ref[...] + hs_ref[...]) + b_ref[...], 0.0)
    h2 = (jnp.dot(h1r, wa_ref[...], preferred_element_type=jnp.float32)
          + jnp.dot(x_ref[...], wb_ref[...], preferred_element_type=jnp.float32))
    out_ref[...] = h2 * dinv[:, None]


def _tc_head_body(q0_ref, q1_ref, hs_ref, dinv_ref, b2_ref, x_ref,
                  f4a_ref, f4b_ref, b4_ref, w5a_ref, w5b_ref, b5_ref, out_ref):
    dinv = dinv_ref[...]
    h2r = jnp.maximum(
        dinv[:, None] * (q0_ref[...] + q1_ref[...] + hs_ref[...]) + b2_ref[...], 0.0)
    hh = jnp.maximum(
        jnp.dot(x_ref[...], f4a_ref[...], preferred_element_type=jnp.float32)
        + jnp.dot(h2r, f4b_ref[...], preferred_element_type=jnp.float32)
        + b4_ref[...], 0.0)
    o = (jnp.dot(x_ref[...], w5a_ref[...], preferred_element_type=jnp.float32)
         + jnp.dot(hh, w5b_ref[...], preferred_element_type=jnp.float32))
    out_ref[...] = jax.nn.sigmoid(o[:, 0] + b5_ref[0])


def _tc_mm1(x_p, W1, degp):
    return pl.pallas_call(
        _tc_mm1_body,
        grid=(_GRID,),
        in_specs=[_row_spec(), _full_spec(F, F),
                  pl.BlockSpec((2, _BLK), lambda i: (0, i))],
        out_specs=[_row_spec(), _vec_spec()],
        out_shape=[jax.ShapeDtypeStruct((NP, F), jnp.float32),
                   jax.ShapeDtypeStruct((NP,), jnp.float32)],
    )(x_p, W1, degp)


def _tc_mid(p0, p1, hs1, dinv, b1, x_p, W2a, W2b):
    return pl.pallas_call(
        _tc_mid_body,
        grid=(_GRID,),
        in_specs=[_row_spec(), _row_spec(), _row_spec(), _vec_spec(),
                  _full_spec(F), _row_spec(), _full_spec(F, F), _full_spec(F, F)],
        out_specs=_row_spec(),
        out_shape=jax.ShapeDtypeStruct((NP, F), jnp.float32),
    )(p0, p1, hs1, dinv, b1, x_p, W2a, W2b)


def _tc_head(q0, q1, hs2, dinv, b2, x_p, f4a, f4b, b4, w5a, w5b, b5):
    return pl.pallas_call(
        _tc_head_body,
        grid=(_GRID,),
        in_specs=[_row_spec(), _row_spec(), _row_spec(), _vec_spec(),
                  _full_spec(F), _row_spec(), _full_spec(F, F), _full_spec(F, F),
                  _full_spec(F), _full_spec(F, 1), _full_spec(F, 1), _full_spec(F)],
        out_specs=_vec_spec(),
        out_shape=jax.ShapeDtypeStruct((NP,), jnp.float32),
    )(q0, q1, hs2, dinv, b2, x_p, f4a, f4b, b4, w5a, w5b, b5)


def kernel(x, edge_index, W1, b1, W2, b2, W3, b3, W4, b4, W5, b5,
           fc1_W, fc1_b, fc2_W, fc2_b, fc3_W, fc3_b, fc4_W, fc4_b,
           fc5_W, fc5_b):
    src = edge_index[:, 0]
    dst = edge_index[:, 1]
    pad = EP - E
    src_p = jnp.concatenate([src, jnp.zeros((pad,), jnp.int32)])
    dst_p = jnp.concatenate([dst, jnp.full((pad,), N, jnp.int32)])
    src3 = src_p.reshape(W, NCH, CH)
    dst3 = dst_p.reshape(W, NCH, CH)

    zeros1 = jnp.zeros((CH,), jnp.float32)
    ones1 = jnp.ones((CH,), jnp.float32)
    zeros2 = jnp.zeros((CH, F), jnp.float32)

    x_p = jnp.pad(x, ((0, NP - N), (0, 0)))

    sc_degree, sc_scatter = _sc_kernels()
    degp = sc_degree(dst3, zeros1, ones1)
    hs1, dinv = _tc_mm1(x_p, W1, degp)

    p = sc_scatter(hs1, src3, dst3, zeros2)
    hs2 = _tc_mid(p[0], p[1], hs1, dinv, b1, x_p, W2[:F], W2[F:])

    q = sc_scatter(hs2, src3, dst3, zeros2)
    out = _tc_head(q[0], q[1], hs2, dinv, b2, x_p,
                   fc4_W[:F], fc4_W[F:], fc4_b, fc5_W[:F], fc5_W[F:],
                   jnp.broadcast_to(fc5_b, (F,)))
    return out[:N]
